# baseline (speedup 1.0000x reference)
import functools

import jax
import jax.numpy as jnp
from jax import lax
from jax.experimental import pallas as pl
from jax.experimental.pallas import tpu as pltpu
from jax.experimental.pallas import tpu_sc as plsc

_VOCAB = 1000000
_DIM = 32
_MAXLEN = 200
_B = 4096
_NCLS = 10
_BN_EPS = 1e-5

_NC = 2
_NS = 16
_NW = _NC * _NS
_BB = _B // _NW


def _sc_gather_transpose(idx3d, table):
    mesh = plsc.VectorSubcoreMesh(core_axis_name="c", subcore_axis_name="s")

    @functools.partial(
        pl.kernel,
        out_type=jax.ShapeDtypeStruct((_MAXLEN, 4, _NW, 1024), jnp.float32),
        mesh=mesh,
        scratch_types=[
            pltpu.VMEM((_MAXLEN, _BB), jnp.int32),
        ]
        + [pltpu.VMEM((_BB, _DIM), jnp.float32) for _ in range(4)]
        + [pltpu.VMEM((4, 1024), jnp.float32) for _ in range(4)]
        + [pltpu.SemaphoreType.DMA for _ in range(8)],
        compiler_params=pltpu.CompilerParams(
            use_tc_tiling_on_sc=False, needs_layout_passes=False
        ),
    )
    def sc_kernel(idx_hbm, tab_hbm, out_hbm, idx_v, rows0, rows1, rows2,
                  rows3, tiles0, tiles1, tiles2, tiles3, sg0, sg1, sg2, sg3,
                  sw0, sw1, sw2, sw3):
        w = lax.axis_index("s") * _NC + lax.axis_index("c")
        pltpu.sync_copy(idx_hbm.at[w], idx_v)

        i16 = lax.iota(jnp.int32, 16)
        sd0 = i16 // 8
        sc0 = (i16 % 8) * 128
        sd1 = sd0 + 2

        rows = (rows0, rows1, rows2, rows3)
        tiles = (tiles0, tiles1, tiles2, tiles3)
        sg = (sg0, sg1, sg2, sg3)
        sw = (sw0, sw1, sw2, sw3)

        def fire_gather(l, j):
            pltpu.async_copy(tab_hbm.at[idx_v.at[l]], rows[j], sg[j])

        def wait_gather(l, j):
            pltpu.make_async_copy(tab_hbm.at[idx_v.at[l]], rows[j],
                                  sg[j]).wait()

        def fire_writes(l, j):
            pltpu.async_copy(tiles[j], out_hbm.at[l, :, w], sw[j])

        def wait_writes(l, j):
            pltpu.make_async_copy(tiles[j], out_hbm.at[l, :, w],
                                  sw[j]).wait()

        for j in range(4):
            fire_gather(j, j)

        @pl.loop(0, _MAXLEN // 4)
        def _(g):
            l0 = 4 * g
            for j in range(4):
                l = l0 + j

                wait_gather(l, j)

                @pl.when(g >= 1)
                def _():
                    wait_writes(l - 4, j)

                @pl.loop(0, _BB, step=8)
                def _(p0):
                    regs = []
                    for t in range(8):
                        p = p0 + t
                        regs.append((p, rows[j][p, pl.ds(0, 16)],
                                     rows[j][p, pl.ds(16, 16)]))
                    for p, r0, r1 in regs:
                        plsc.store_scatter(tiles[j], [sd0, sc0 + p], r0)
                        plsc.store_scatter(tiles[j], [sd1, sc0 + p], r1)

                @pl.when(g < _MAXLEN // 4 - 1)
                def _():
                    fire_gather(l + 4, j)

                fire_writes(l, j)

        for j in range(4):
            wait_writes(_MAXLEN - 4 + j, j)

    return sc_kernel(idx3d, table)


def _tc_reduce_body(raw_ref, avg_ref):
    i = pl.program_id(0)

    @pl.when(i == 0)
    def _():
        avg_ref[...] = jnp.zeros_like(avg_ref)

    avg_ref[...] += jnp.sum(raw_ref[...], axis=0)

    @pl.when(i == pl.num_programs(0) - 1)
    def _():
        avg_ref[...] *= 1.0 / _MAXLEN


def _tc_reduce(raw5):
    lblk = 8
    return pl.pallas_call(
        _tc_reduce_body,
        grid=(_MAXLEN // lblk,),
        in_specs=[
            pl.BlockSpec((lblk, 4, _NW, 8, 128), lambda i: (i, 0, 0, 0, 0))
        ],
        out_specs=pl.BlockSpec((4, _NW, 8, 128), lambda i: (0, 0, 0, 0)),
        out_shape=jax.ShapeDtypeStruct((4, _NW, 8, 128), jnp.float32),
    )(raw5)


def _tc_head_body(xavg_ref, gamma_ref, beta_ref, mean_ref, var_ref, fcw_ref,
                  fcb_ref, bn_ref, final_ref):
    x_avg = xavg_ref[...]
    bn = (x_avg - mean_ref[...]) / jnp.sqrt(var_ref[...] + _BN_EPS) \
        * gamma_ref[...] + beta_ref[...]
    bn_ref[...] = bn
    final_ref[...] = lax.dot_general(
        bn, fcw_ref[...],
        dimension_numbers=(((1,), (1,)), ((), ())),
        preferred_element_type=jnp.float32,
    ) + fcb_ref[...]


def _tc_head(x_avg, bn_gamma, bn_beta, bn_mean, bn_var, fc_w, fc_b):
    f32 = jnp.float32
    return pl.pallas_call(
        _tc_head_body,
        out_shape=[
            jax.ShapeDtypeStruct((_B, _DIM), f32),
            jax.ShapeDtypeStruct((_B, _NCLS), f32),
        ],
    )(
        x_avg,
        bn_gamma.reshape(1, _DIM),
        bn_beta.reshape(1, _DIM),
        bn_mean.reshape(1, _DIM),
        bn_var.reshape(1, _DIM),
        fc_w,
        fc_b.reshape(1, _NCLS),
    )


def kernel(word_idx, table, bn_gamma, bn_beta, bn_mean, bn_var, fc_w, fc_b):
    idx3d = word_idx.reshape(_NW, _BB, _MAXLEN).transpose(0, 2, 1)
    out = _sc_gather_transpose(idx3d, table)
    raw5 = out.reshape(_MAXLEN, 4, _NW, 8, 128)
    raw_feature = raw5.transpose(2, 4, 0, 1, 3).reshape(_B, _MAXLEN, _DIM)
    avg4 = _tc_reduce(raw5)
    x_avg = avg4.transpose(1, 3, 0, 2).reshape(_B, _DIM)
    x_avg_bn, x_final = _tc_head(
        x_avg, bn_gamma, bn_beta, bn_mean, bn_var, fc_w, fc_b
    )
    return (x_final, x_avg_bn, x_avg, raw_feature)


# speedup vs baseline: 1.0019x; 1.0019x over previous
import functools

import jax
import jax.numpy as jnp
from jax import lax
from jax.experimental import pallas as pl
from jax.experimental.pallas import tpu as pltpu
from jax.experimental.pallas import tpu_sc as plsc

_VOCAB = 1000000
_DIM = 32
_MAXLEN = 200
_B = 4096
_NCLS = 10
_BN_EPS = 1e-5

_NC = 2
_NS = 16
_NW = _NC * _NS
_BB = _B // _NW

_VB = 1024
_NBLK = 977
_VPAD = _NBLK * _VB


def _tc_detile_body(tt_ref, out_ref, scr_ref):
    t = tt_ref[...].T
    scr_ref[...] = t.reshape(_VB // 4, 4, _DIM)
    parts = [scr_ref[:, c, :] for c in range(4)]
    out_ref[...] = jnp.concatenate(parts, axis=1)


def _tc_detile(tableT):
    return pl.pallas_call(
        _tc_detile_body,
        grid=(_NBLK,),
        in_specs=[pl.BlockSpec((_DIM, _VB), lambda i: (0, i))],
        out_specs=pl.BlockSpec((_VB // 4, 128), lambda i: (i, 0)),
        out_shape=jax.ShapeDtypeStruct((_VPAD // 4, 128), jnp.float32),
        scratch_shapes=[pltpu.VMEM((_VB // 4, 4, _DIM), jnp.float32)],
    )(tableT)


def _sc_gather_transpose(idx3d, table_lin):
    mesh = plsc.VectorSubcoreMesh(core_axis_name="c", subcore_axis_name="s")

    @functools.partial(
        pl.kernel,
        out_type=jax.ShapeDtypeStruct((_MAXLEN, 4, _NW, 1024), jnp.float32),
        mesh=mesh,
        scratch_types=[
            pltpu.VMEM((_MAXLEN, _BB), jnp.int32),
        ]
        + [pltpu.VMEM((_BB, _DIM), jnp.float32) for _ in range(4)]
        + [pltpu.VMEM((4, 1024), jnp.float32) for _ in range(4)]
        + [pltpu.SemaphoreType.DMA for _ in range(8)],
        compiler_params=pltpu.CompilerParams(
            use_tc_tiling_on_sc=False, needs_layout_passes=False
        ),
    )
    def sc_kernel(idx_hbm, tab_hbm, out_hbm, idx_v, rows0, rows1, rows2,
                  rows3, tiles0, tiles1, tiles2, tiles3, sg0, sg1, sg2, sg3,
                  sw0, sw1, sw2, sw3):
        w = lax.axis_index("s") * _NC + lax.axis_index("c")
        pltpu.sync_copy(idx_hbm.at[w], idx_v)

        i16 = lax.iota(jnp.int32, 16)
        sd0 = i16 // 8
        sc0 = (i16 % 8) * 128
        sd1 = sd0 + 2

        rows = (rows0, rows1, rows2, rows3)
        tiles = (tiles0, tiles1, tiles2, tiles3)
        sg = (sg0, sg1, sg2, sg3)
        sw = (sw0, sw1, sw2, sw3)

        def fire_gather(l, j):
            pltpu.async_copy(tab_hbm.at[idx_v.at[l]], rows[j], sg[j])

        def wait_gather(l, j):
            pltpu.make_async_copy(tab_hbm.at[idx_v.at[l]], rows[j],
                                  sg[j]).wait()

        def fire_writes(l, j):
            pltpu.async_copy(tiles[j], out_hbm.at[l, :, w], sw[j])

        def wait_writes(l, j):
            pltpu.make_async_copy(tiles[j], out_hbm.at[l, :, w],
                                  sw[j]).wait()

        for j in range(4):
            fire_gather(j, j)

        @pl.loop(0, _MAXLEN // 4)
        def _(g):
            l0 = 4 * g
            for j in range(4):
                l = l0 + j

                wait_gather(l, j)

                @pl.when(g >= 1)
                def _():
                    wait_writes(l - 4, j)

                @pl.loop(0, _BB, step=8)
                def _(p0):
                    regs = []
                    for t in range(8):
                        p = p0 + t
                        regs.append((p, rows[j][p, pl.ds(0, 16)],
                                     rows[j][p, pl.ds(16, 16)]))
                    for p, r0, r1 in regs:
                        plsc.store_scatter(tiles[j], [sd0, sc0 + p], r0)
                        plsc.store_scatter(tiles[j], [sd1, sc0 + p], r1)

                @pl.when(g < _MAXLEN // 4 - 1)
                def _():
                    fire_gather(l + 4, j)

                fire_writes(l, j)

        for j in range(4):
            wait_writes(_MAXLEN - 4 + j, j)

    return sc_kernel(idx3d, table_lin)


def _tc_reduce_body(raw_ref, avg_ref):
    i = pl.program_id(0)

    @pl.when(i == 0)
    def _():
        avg_ref[...] = jnp.zeros_like(avg_ref)

    avg_ref[...] += jnp.sum(raw_ref[...], axis=0)

    @pl.when(i == pl.num_programs(0) - 1)
    def _():
        avg_ref[...] *= 1.0 / _MAXLEN


def _tc_reduce(raw5):
    lblk = 8
    return pl.pallas_call(
        _tc_reduce_body,
        grid=(_MAXLEN // lblk,),
        in_specs=[
            pl.BlockSpec((lblk, 4, _NW, 8, 128), lambda i: (i, 0, 0, 0, 0))
        ],
        out_specs=pl.BlockSpec((4, _NW, 8, 128), lambda i: (0, 0, 0, 0)),
        out_shape=jax.ShapeDtypeStruct((4, _NW, 8, 128), jnp.float32),
    )(raw5)


def _tc_head_body(xavg_ref, gamma_ref, beta_ref, mean_ref, var_ref, fcw_ref,
                  fcb_ref, bn_ref, final_ref):
    x_avg = xavg_ref[...]
    bn = (x_avg - mean_ref[...]) / jnp.sqrt(var_ref[...] + _BN_EPS) \
        * gamma_ref[...] + beta_ref[...]
    bn_ref[...] = bn
    final_ref[...] = lax.dot_general(
        bn, fcw_ref[...],
        dimension_numbers=(((1,), (1,)), ((), ())),
        preferred_element_type=jnp.float32,
    ) + fcb_ref[...]


def _tc_head(x_avg, bn_gamma, bn_beta, bn_mean, bn_var, fc_w, fc_b):
    f32 = jnp.float32
    return pl.pallas_call(
        _tc_head_body,
        out_shape=[
            jax.ShapeDtypeStruct((_B, _DIM), f32),
            jax.ShapeDtypeStruct((_B, _NCLS), f32),
        ],
    )(
        x_avg,
        bn_gamma.reshape(1, _DIM),
        bn_beta.reshape(1, _DIM),
        bn_mean.reshape(1, _DIM),
        bn_var.reshape(1, _DIM),
        fc_w,
        fc_b.reshape(1, _NCLS),
    )


def kernel(word_idx, table, bn_gamma, bn_beta, bn_mean, bn_var, fc_w, fc_b):
    idx3d = word_idx.reshape(_NW, _BB, _MAXLEN).transpose(0, 2, 1)
    table_lin = _tc_detile(table.T).reshape(_VPAD, _DIM)
    out = _sc_gather_transpose(idx3d, table_lin)
    raw5 = out.reshape(_MAXLEN, 4, _NW, 8, 128)
    raw_feature = raw5.transpose(2, 4, 0, 1, 3).reshape(_B, _MAXLEN, _DIM)
    avg4 = _tc_reduce(raw5)
    x_avg = avg4.transpose(1, 3, 0, 2).reshape(_B, _DIM)
    x_avg_bn, x_final = _tc_head(
        x_avg, bn_gamma, bn_beta, bn_mean, bn_var, fc_w, fc_b
    )
    return (x_final, x_avg_bn, x_avg, raw_feature)


# speedup vs baseline: 1.2260x; 1.2237x over previous
import functools

import jax
import jax.numpy as jnp
from jax import lax
from jax.experimental import pallas as pl
from jax.experimental.pallas import tpu as pltpu
from jax.experimental.pallas import tpu_sc as plsc

_VOCAB = 1000000
_DIM = 32
_MAXLEN = 200
_B = 4096
_NCLS = 10
_BN_EPS = 1e-5

_NC = 2
_NS = 16
_NW = _NC * _NS
_BB = _B // _NW


def _sc_gather_transpose(idx3d, table_lin):
    mesh = plsc.VectorSubcoreMesh(core_axis_name="c", subcore_axis_name="s")

    @functools.partial(
        pl.kernel,
        out_type=jax.ShapeDtypeStruct((_MAXLEN, 4, _NW, 1024), jnp.float32),
        mesh=mesh,
        scratch_types=[
            pltpu.VMEM((_MAXLEN, _BB), jnp.int32),
        ]
        + [pltpu.VMEM((_BB, _DIM), jnp.float32) for _ in range(4)]
        + [pltpu.VMEM((4, 1024), jnp.float32) for _ in range(4)]
        + [pltpu.SemaphoreType.DMA for _ in range(8)],
        compiler_params=pltpu.CompilerParams(
            use_tc_tiling_on_sc=False, needs_layout_passes=False
        ),
    )
    def sc_kernel(idx_hbm, tab_hbm, out_hbm, idx_v, rows0, rows1, rows2,
                  rows3, tiles0, tiles1, tiles2, tiles3, sg0, sg1, sg2, sg3,
                  sw0, sw1, sw2, sw3):
        w = lax.axis_index("s") * _NC + lax.axis_index("c")
        pltpu.sync_copy(idx_hbm.at[w], idx_v)

        i16 = lax.iota(jnp.int32, 16)
        sd0 = i16 // 8
        sc0 = (i16 % 8) * 128
        sd1 = sd0 + 2

        rows = (rows0, rows1, rows2, rows3)
        tiles = (tiles0, tiles1, tiles2, tiles3)
        sg = (sg0, sg1, sg2, sg3)
        sw = (sw0, sw1, sw2, sw3)

        def fire_gather(l, j):
            pltpu.async_copy(tab_hbm.at[idx_v.at[l]], rows[j], sg[j])

        def wait_gather(l, j):
            pltpu.make_async_copy(tab_hbm.at[idx_v.at[l]], rows[j],
                                  sg[j]).wait()

        def fire_writes(l, j):
            pltpu.async_copy(tiles[j], out_hbm.at[l, :, w], sw[j])

        def wait_writes(l, j):
            pltpu.make_async_copy(tiles[j], out_hbm.at[l, :, w],
                                  sw[j]).wait()

        for j in range(4):
            fire_gather(j, j)

        @pl.loop(0, _MAXLEN // 4)
        def _(g):
            l0 = 4 * g
            for j in range(4):
                l = l0 + j

                wait_gather(l, j)

                @pl.when(g >= 1)
                def _():
                    wait_writes(l - 4, j)

                @pl.loop(0, _BB, step=8)
                def _(p0):
                    regs = []
                    for t in range(8):
                        p = p0 + t
                        regs.append((p, rows[j][p, pl.ds(0, 16)],
                                     rows[j][p, pl.ds(16, 16)]))
                    for p, r0, r1 in regs:
                        plsc.store_scatter(tiles[j], [sd0, sc0 + p], r0)
                        plsc.store_scatter(tiles[j], [sd1, sc0 + p], r1)

                @pl.when(g < _MAXLEN // 4 - 1)
                def _():
                    fire_gather(l + 4, j)

                fire_writes(l, j)

        for j in range(4):
            wait_writes(_MAXLEN - 4 + j, j)

    return sc_kernel(idx3d, table_lin)


def _tc_reduce_body(raw_ref, avg_ref):
    i = pl.program_id(0)

    @pl.when(i == 0)
    def _():
        avg_ref[...] = jnp.zeros_like(avg_ref)

    avg_ref[...] += jnp.sum(raw_ref[...], axis=0)

    @pl.when(i == pl.num_programs(0) - 1)
    def _():
        avg_ref[...] *= 1.0 / _MAXLEN


def _tc_reduce(raw5):
    lblk = 8
    return pl.pallas_call(
        _tc_reduce_body,
        grid=(_MAXLEN // lblk,),
        in_specs=[
            pl.BlockSpec((lblk, 4, _NW, 8, 128), lambda i: (i, 0, 0, 0, 0))
        ],
        out_specs=pl.BlockSpec((4, _NW, 8, 128), lambda i: (0, 0, 0, 0)),
        out_shape=jax.ShapeDtypeStruct((4, _NW, 8, 128), jnp.float32),
    )(raw5)


def _tc_head_body(xavg_ref, gamma_ref, beta_ref, mean_ref, var_ref, fcw_ref,
                  fcb_ref, bn_ref, final_ref):
    x_avg = xavg_ref[...]
    bn = (x_avg - mean_ref[...]) / jnp.sqrt(var_ref[...] + _BN_EPS) \
        * gamma_ref[...] + beta_ref[...]
    bn_ref[...] = bn
    final_ref[...] = lax.dot_general(
        bn, fcw_ref[...],
        dimension_numbers=(((1,), (1,)), ((), ())),
        preferred_element_type=jnp.float32,
    ) + fcb_ref[...]


def _tc_head(x_avg, bn_gamma, bn_beta, bn_mean, bn_var, fc_w, fc_b):
    f32 = jnp.float32
    return pl.pallas_call(
        _tc_head_body,
        out_shape=[
            jax.ShapeDtypeStruct((_B, _DIM), f32),
            jax.ShapeDtypeStruct((_B, _NCLS), f32),
        ],
    )(
        x_avg,
        bn_gamma.reshape(1, _DIM),
        bn_beta.reshape(1, _DIM),
        bn_mean.reshape(1, _DIM),
        bn_var.reshape(1, _DIM),
        fc_w,
        fc_b.reshape(1, _NCLS),
    )


def kernel(word_idx, table, bn_gamma, bn_beta, bn_mean, bn_var, fc_w, fc_b):
    idx3d = word_idx.reshape(_NW, _BB, _MAXLEN).transpose(0, 2, 1)
    out = _sc_gather_transpose(idx3d, table)
    raw5 = out.reshape(_MAXLEN, 4, _NW, 8, 128)
    raw_feature = raw5.transpose(2, 4, 0, 1, 3).reshape(_B, _MAXLEN, _DIM)
    avg4 = _tc_reduce(raw5)
    x_avg = avg4.transpose(1, 3, 0, 2).reshape(_B, _DIM)
    x_avg_bn, x_final = _tc_head(
        x_avg, bn_gamma, bn_beta, bn_mean, bn_var, fc_w, fc_b
    )
    return (x_final, x_avg_bn, x_avg, raw_feature)
